# Initial kernel scaffold; baseline (speedup 1.0000x reference)
import jax
import jax.numpy as jnp
from jax.experimental import pallas as pl


def kernel(x, table):
    raise NotImplementedError("write your pallas kernel here")


# speedup vs baseline: 1.1389x; 1.1389x over previous
import functools

import jax
import jax.numpy as jnp
from jax import lax
from jax.experimental import pallas as pl
from jax.experimental.pallas import tpu as pltpu
from jax.experimental.pallas import tpu_sc as plsc

_D = 128
_NC = 2
_NS = 16
_NW = _NC * _NS
_CHUNK = 128


@functools.lru_cache(maxsize=None)
def _make_gather(n_chunks: int):
    b_total = _NW * n_chunks * _CHUNK
    mesh = plsc.VectorSubcoreMesh(core_axis_name="c", subcore_axis_name="s")

    @functools.partial(
        pl.kernel,
        mesh=mesh,
        out_type=jax.ShapeDtypeStruct((_NW, n_chunks, _CHUNK, _D),
                                      jnp.float32),
        scratch_types=[
            pltpu.VMEM((n_chunks, _CHUNK), jnp.int32),
            pltpu.VMEM((_CHUNK, _D), jnp.float32),
            pltpu.SemaphoreType.DMA,
        ],
    )
    def k(idx_hbm, table_hbm, out_hbm, idx_v, rows_v, sem):
        wid = lax.axis_index("s") * _NC + lax.axis_index("c")
        pltpu.sync_copy(idx_hbm.at[wid], idx_v)

        def body(j, _):
            pltpu.async_copy(table_hbm.at[idx_v.at[j]], rows_v, sem).wait()
            pltpu.sync_copy(rows_v, out_hbm.at[wid, j])
            return 0

        lax.fori_loop(0, n_chunks, body, 0)

    del b_total
    return k


def kernel(x, table):
    b, s = x.shape
    d = table.shape[1]
    n_total = b * s
    n_chunks = n_total // (_NW * _CHUNK)
    idx = x.reshape(-1).astype(jnp.int32).reshape(_NW, n_chunks, _CHUNK)
    out = _make_gather(n_chunks)(idx, table)
    return out.reshape(b, s, d)


# speedup vs baseline: 1.2843x; 1.1277x over previous
import functools

import jax
import jax.numpy as jnp
from jax import lax
from jax.experimental import pallas as pl
from jax.experimental.pallas import tpu as pltpu
from jax.experimental.pallas import tpu_sc as plsc

_D = 128
_NC = 2
_NS = 16
_NW = _NC * _NS
_CHUNK = 128


_NBUF = 5


@functools.lru_cache(maxsize=None)
def _make_gather(n_chunks: int):
    assert n_chunks % _NBUF == 0
    mesh = plsc.VectorSubcoreMesh(core_axis_name="c", subcore_axis_name="s")

    @functools.partial(
        pl.kernel,
        mesh=mesh,
        out_type=jax.ShapeDtypeStruct((_NW, n_chunks, _CHUNK, _D),
                                      jnp.float32),
        scratch_types=[
            pltpu.VMEM((n_chunks, _CHUNK), jnp.int32),
            *[pltpu.VMEM((_CHUNK, _D), jnp.float32) for _ in range(_NBUF)],
            *[pltpu.SemaphoreType.DMA for _ in range(2 * _NBUF)],
        ],
    )
    def k(idx_hbm, table_hbm, out_hbm, idx_v, *bufs_and_sems):
        bufs = bufs_and_sems[:_NBUF]
        gsem = bufs_and_sems[_NBUF:2 * _NBUF]
        psem = bufs_and_sems[2 * _NBUF:]
        wid = lax.axis_index("s") * _NC + lax.axis_index("c")
        pltpu.sync_copy(idx_hbm.at[wid], idx_v)

        def gather(j, b):
            return pltpu.make_async_copy(
                table_hbm.at[idx_v.at[j]], bufs[b], gsem[b])

        def put(j, b):
            return pltpu.make_async_copy(bufs[b], out_hbm.at[wid, j], psem[b])

        for b in range(_NBUF):
            gather(b, b).start()

        def group(g, _):
            j0 = g * _NBUF
            for b in range(_NBUF):
                j = j0 + b
                gather(j, b).wait()
                put(j, b).start()
                @pl.when(j + _NBUF < n_chunks)
                def _():
                    put(j, b).wait()
                    gather(j + _NBUF, b).start()
            return 0

        lax.fori_loop(0, n_chunks // _NBUF, group, 0)
        for b in range(_NBUF):
            put(n_chunks - _NBUF + b, b).wait()

    return k


def kernel(x, table):
    b, s = x.shape
    d = table.shape[1]
    n_total = b * s
    n_chunks = n_total // (_NW * _CHUNK)
    idx = x.reshape(-1).astype(jnp.int32).reshape(_NW, n_chunks, _CHUNK)
    out = _make_gather(n_chunks)(idx, table)
    return out.reshape(b, s, d)


# speedup vs baseline: 2.2911x; 1.7839x over previous
import functools

import jax
import jax.numpy as jnp
from jax import lax
from jax.experimental import pallas as pl
from jax.experimental.pallas import tpu as pltpu
from jax.experimental.pallas import tpu_sc as plsc

_NC = 2
_NS = 16
_NW = _NC * _NS
_RPC = 2
_NBUF = 4


@functools.lru_cache(maxsize=None)
def _make_gather(b: int, s: int, d: int):
    rows_per_w = b // _NW
    n_chunks = rows_per_w // _RPC
    chunk = _RPC * s
    assert chunk <= 128 and n_chunks % _NBUF == 0
    mesh = plsc.VectorSubcoreMesh(core_axis_name="c", subcore_axis_name="s")

    @functools.partial(
        pl.kernel,
        mesh=mesh,
        out_type=jax.ShapeDtypeStruct((b, s, d), jnp.float32),
        scratch_types=[
            pltpu.VMEM((n_chunks, chunk), jnp.int32),
            *[pltpu.VMEM((chunk, d), jnp.float32) for _ in range(_NBUF)],
            *[pltpu.SemaphoreType.DMA for _ in range(2 * _NBUF)],
        ],
    )
    def k(idx_hbm, table_hbm, out_hbm, idx_v, *bufs_and_sems):
        bufs = bufs_and_sems[:_NBUF]
        gsem = bufs_and_sems[_NBUF:2 * _NBUF]
        psem = bufs_and_sems[2 * _NBUF:]
        wid = lax.axis_index("s") * _NC + lax.axis_index("c")
        row0 = wid * rows_per_w
        pltpu.sync_copy(idx_hbm.at[wid], idx_v)

        def gather(j, bf):
            return pltpu.make_async_copy(
                table_hbm.at[idx_v.at[j]], bufs[bf], gsem[bf])

        def puts(j, bf):
            return [
                pltpu.make_async_copy(
                    bufs[bf].at[pl.ds(r * s, s)],
                    out_hbm.at[row0 + j * _RPC + r],
                    psem[bf])
                for r in range(_RPC)
            ]

        for bf in range(_NBUF):
            gather(bf, bf).start()

        def group(g, _):
            j0 = g * _NBUF
            for bf in range(_NBUF):
                j = j0 + bf
                gather(j, bf).wait()
                for p in puts(j, bf):
                    p.start()
                @pl.when(j + _NBUF < n_chunks)
                def _():
                    for p in puts(j, bf):
                        p.wait()
                    gather(j + _NBUF, bf).start()
            return 0

        lax.fori_loop(0, n_chunks // _NBUF, group, 0)
        for bf in range(_NBUF):
            for p in puts(n_chunks - _NBUF + bf, bf):
                p.wait()

    return k


def kernel(x, table):
    b, s = x.shape
    d = table.shape[1]
    rows_per_w = b // _NW
    idx = x.astype(jnp.int32).reshape(_NW, rows_per_w // _RPC, _RPC * s)
    return _make_gather(b, s, d)(idx, table)
